# Initial kernel scaffold; baseline (speedup 1.0000x reference)
import jax
import jax.numpy as jnp
from jax.experimental import pallas as pl


def kernel(loc_p, conf_p, priors, targets):
    raise NotImplementedError("write your pallas kernel here")


# speedup vs baseline: 80.5446x; 80.5446x over previous
import functools

import jax
import jax.numpy as jnp
from jax import lax
from jax.experimental import pallas as pl
from jax.experimental.pallas import tpu as pltpu

_MATCH_OVERLAP = 0.35
_NEG_POS_RATE = 3
_VAR0, _VAR1 = 0.1, 0.2
_CONF_GAIN = 1.0

_T = 32
_P = 32768
_R, _C = 8, 4096
_BIG_I32 = 2147483647


def _softplus(x):
    return jnp.maximum(x, 0.0) + jnp.log1p(jnp.exp(-jnp.abs(x)))


def _loss_kernel(targets_ref, locr_ref, conf_ref, pf_ref, out_ref,
                 bto_ref, matched_ref):
    t5 = targets_ref[0]
    tx1 = t5[:, 0:1]
    ty1 = t5[:, 1:2]
    tx2 = t5[:, 2:3]
    ty2 = t5[:, 3:4]
    area_t = (tx2 - tx1) * (ty2 - ty1)
    truths4 = t5[:, 0:4]

    iota_t = lax.broadcasted_iota(jnp.int32, (_T, 1), 0)
    iota_c = lax.broadcasted_iota(jnp.int32, (1, _C), 1)

    run_max = jnp.full((_T, 1), -1.0, dtype=jnp.float32)
    run_idx = jnp.zeros((_T, 1), dtype=jnp.int32)

    for c in range(_R):
        px1 = pf_ref[4, c, :][None, :]
        py1 = pf_ref[5, c, :][None, :]
        px2 = pf_ref[6, c, :][None, :]
        py2 = pf_ref[7, c, :][None, :]
        area_p = pf_ref[8, c, :][None, :]

        iw = jnp.maximum(jnp.minimum(tx2, px2) - jnp.maximum(tx1, px1), 0.0)
        ih = jnp.maximum(jnp.minimum(ty2, py2) - jnp.maximum(ty1, py1), 0.0)
        inter = iw * ih
        iou = inter / (area_t + area_p - inter)

        bto_c = jnp.max(iou, axis=0, keepdims=True)
        bti_c = jnp.min(jnp.where(iou == bto_c, iota_t, _BIG_I32),
                        axis=0, keepdims=True)
        oh = (bti_c == iota_t).astype(jnp.float32)
        matched_c = lax.dot_general(truths4, oh, (((0,), (0,)), ((), ())),
                                    preferred_element_type=jnp.float32)
        bto_ref[c, :] = bto_c[0]
        matched_ref[:, c, :] = matched_c

        cmax = jnp.max(iou, axis=1, keepdims=True)
        carg = jnp.min(jnp.where(iou == cmax, iota_c, _BIG_I32),
                       axis=1, keepdims=True) + c * _C
        upd = cmax > run_max
        run_max = jnp.where(upd, cmax, run_max)
        run_idx = jnp.where(upd, carg, run_idx)

    ri8 = lax.broadcasted_iota(jnp.int32, (8, 128), 0)
    li8 = lax.broadcasted_iota(jnp.int32, (8, 128), 1)
    for t in range(_T):
        p = run_idx[t, 0]
        r = p // _C
        l = p - r * _C
        lane_base = pl.multiple_of((l // 128) * 128, 128)
        l_off = l - lane_base
        m = (ri8 == r) & (li8 == l_off)
        tile = bto_ref[:, pl.ds(lane_base, 128)]
        bto_ref[:, pl.ds(lane_base, 128)] = jnp.where(m, 2.0, tile)
        mtile = matched_ref[:, :, pl.ds(lane_base, 128)]
        coords = truths4[t, :].reshape(4, 1, 1)
        matched_ref[:, :, pl.ds(lane_base, 128)] = jnp.where(
            m[None], coords, mtile)

    bto = bto_ref[:, :]
    pos = bto >= _MATCH_OVERLAP
    posf = pos.astype(jnp.float32)
    pos_cnt = jnp.sum(posf)

    cx = pf_ref[0]
    cy = pf_ref[1]
    pw = pf_ref[2]
    ph = pf_ref[3]
    mx1 = matched_ref[0]
    my1 = matched_ref[1]
    mx2 = matched_ref[2]
    my2 = matched_ref[3]

    ltx = ((mx1 + mx2) * 0.5 - cx) / (_VAR0 * pw)
    lty = ((my1 + my2) * 0.5 - cy) / (_VAR0 * ph)
    ltw = jnp.log((mx2 - mx1) / pw) / _VAR1
    lth = jnp.log((my2 - my1) / ph) / _VAR1

    def sl1(d):
        a = jnp.abs(d)
        return jnp.where(a < 1.0, 0.5 * d * d, a - 0.5)

    l0 = sl1(locr_ref[0, 0] - ltx) + sl1(locr_ref[0, 1] - lty)
    l1 = sl1(locr_ref[0, 2] - ltw) + sl1(locr_ref[0, 3] - lth)
    loc_partial = jnp.sum((l0 + l1) * posf)

    x = conf_ref[0]
    sp = _softplus(x)
    pos_conf_partial = jnp.sum(posf * (sp - x))
    neg_clm = jnp.where(pos, 0.0, sp)

    k = jnp.minimum(jnp.sum(pos.astype(jnp.int32)) * _NEG_POS_RATE, _P)
    keys = neg_clm.view(jnp.int32)

    thr = jnp.int32(0)
    for bit in range(30, -1, -1):
        cand = thr | jnp.int32(1 << bit)
        cnt = jnp.sum((keys >= cand).astype(jnp.int32))
        thr = jnp.where(cnt >= k, cand, thr)
    above = keys > thr
    cnt_gt = jnp.sum(above.astype(jnp.int32))
    sum_gt = jnp.sum(jnp.where(above, neg_clm, 0.0))
    vk = thr.view(jnp.float32)
    neg_partial = jnp.where(
        k > 0, sum_gt + (k - cnt_gt).astype(jnp.float32) * vk, 0.0)

    lane = lax.broadcasted_iota(jnp.int32, (1, 128), 1)
    row = (jnp.where(lane == 0, loc_partial, 0.0)
           + jnp.where(lane == 1, pos_conf_partial, 0.0)
           + jnp.where(lane == 2, neg_partial, 0.0)
           + jnp.where(lane == 3, pos_cnt, 0.0))
    out_ref[0] = row


def kernel(loc_p, conf_p, priors, targets):
    B, P = conf_p.shape
    pcx, pcy = priors[:, 0], priors[:, 1]
    pww, phh = priors[:, 2], priors[:, 3]
    px1 = pcx - pww * 0.5
    py1 = pcy - phh * 0.5
    px2 = pcx + pww * 0.5
    py2 = pcy + phh * 0.5
    area = pww * phh
    pf = jnp.stack([pcx, pcy, pww, phh, px1, py1, px2, py2, area]
                   ).reshape(9, _R, _C)

    locr = loc_p.transpose(0, 2, 1).reshape(B, 4, _R, _C)
    confr = conf_p.reshape(B, _R, _C)

    out = pl.pallas_call(
        _loss_kernel,
        grid=(B,),
        in_specs=[
            pl.BlockSpec((1, _T, 5), lambda b: (b, 0, 0)),
            pl.BlockSpec((1, 4, _R, _C), lambda b: (b, 0, 0, 0)),
            pl.BlockSpec((1, _R, _C), lambda b: (b, 0, 0)),
            pl.BlockSpec((9, _R, _C), lambda b: (0, 0, 0)),
        ],
        out_specs=pl.BlockSpec((1, 1, 128), lambda b: (b, 0, 0)),
        out_shape=jax.ShapeDtypeStruct((B, 1, 128), jnp.float32),
        scratch_shapes=[
            pltpu.VMEM((_R, _C), jnp.float32),
            pltpu.VMEM((4, _R, _C), jnp.float32),
        ],
    )(targets, locr, confr, pf)

    parts = jnp.sum(out[:, 0, 0:4], axis=0)
    pos_sum = jnp.maximum(parts[3], 1.0)
    loc_loss = parts[0] / pos_sum
    conf_loss = _CONF_GAIN * (parts[1] + parts[2]) / pos_sum
    return (loc_loss, conf_loss)


# speedup vs baseline: 80.6251x; 1.0010x over previous
import functools

import jax
import jax.numpy as jnp
from jax import lax
from jax.experimental import pallas as pl
from jax.experimental.pallas import tpu as pltpu

_MATCH_OVERLAP = 0.35
_NEG_POS_RATE = 3
_VAR0, _VAR1 = 0.1, 0.2
_CONF_GAIN = 1.0

_T = 32
_P = 32768
_R, _C = 8, 4096
_BIG_I32 = 2147483647


def _softplus(x):
    return jnp.maximum(x, 0.0) + jnp.log1p(jnp.exp(-jnp.abs(x)))


def _loss_kernel(targets_ref, locr_ref, conf_ref, pf_ref, out_ref,
                 bto_ref, matched_ref):
    t5 = targets_ref[0]
    tx1 = t5[:, 0:1]
    ty1 = t5[:, 1:2]
    tx2 = t5[:, 2:3]
    ty2 = t5[:, 3:4]
    area_t = (tx2 - tx1) * (ty2 - ty1)
    truths4 = t5[:, 0:4]

    iota_t = lax.broadcasted_iota(jnp.int32, (_T, 1), 0)
    iota_c = lax.broadcasted_iota(jnp.int32, (1, _C), 1)

    run_max = jnp.full((_T, 1), -1.0, dtype=jnp.float32)
    run_idx = jnp.zeros((_T, 1), dtype=jnp.int32)

    for c in range(_R):
        px1 = pf_ref[4, c, :][None, :]
        py1 = pf_ref[5, c, :][None, :]
        px2 = pf_ref[6, c, :][None, :]
        py2 = pf_ref[7, c, :][None, :]
        area_p = pf_ref[8, c, :][None, :]

        iw = jnp.maximum(jnp.minimum(tx2, px2) - jnp.maximum(tx1, px1), 0.0)
        ih = jnp.maximum(jnp.minimum(ty2, py2) - jnp.maximum(ty1, py1), 0.0)
        inter = iw * ih
        iou = inter / (area_t + area_p - inter)

        bto_c = jnp.max(iou, axis=0, keepdims=True)
        bti_c = jnp.min(jnp.where(iou == bto_c, iota_t, _BIG_I32),
                        axis=0, keepdims=True)
        oh = (bti_c == iota_t).astype(jnp.float32)
        matched_c = lax.dot_general(truths4, oh, (((0,), (0,)), ((), ())),
                                    preferred_element_type=jnp.float32)
        bto_ref[c, :] = bto_c[0]
        matched_ref[:, c, :] = matched_c

        cmax = jnp.max(iou, axis=1, keepdims=True)
        carg = jnp.min(jnp.where(iou == cmax, iota_c, _BIG_I32),
                       axis=1, keepdims=True) + c * _C
        upd = cmax > run_max
        run_max = jnp.where(upd, cmax, run_max)
        run_idx = jnp.where(upd, carg, run_idx)

    ri8 = lax.broadcasted_iota(jnp.int32, (8, 128), 0)
    li8 = lax.broadcasted_iota(jnp.int32, (8, 128), 1)
    for t in range(_T):
        p = run_idx[t, 0]
        r = p // _C
        l = p - r * _C
        lane_base = pl.multiple_of((l // 128) * 128, 128)
        l_off = l - lane_base
        m = (ri8 == r) & (li8 == l_off)
        tile = bto_ref[:, pl.ds(lane_base, 128)]
        bto_ref[:, pl.ds(lane_base, 128)] = jnp.where(m, 2.0, tile)
        mtile = matched_ref[:, :, pl.ds(lane_base, 128)]
        coords = truths4[t, :].reshape(4, 1, 1)
        matched_ref[:, :, pl.ds(lane_base, 128)] = jnp.where(
            m[None], coords, mtile)

    bto = bto_ref[:, :]
    pos = bto >= _MATCH_OVERLAP
    posf = pos.astype(jnp.float32)
    pos_cnt = jnp.sum(posf)

    cx = pf_ref[0]
    cy = pf_ref[1]
    pw = pf_ref[2]
    ph = pf_ref[3]
    mx1 = matched_ref[0]
    my1 = matched_ref[1]
    mx2 = matched_ref[2]
    my2 = matched_ref[3]

    ltx = ((mx1 + mx2) * 0.5 - cx) / (_VAR0 * pw)
    lty = ((my1 + my2) * 0.5 - cy) / (_VAR0 * ph)
    ltw = jnp.log((mx2 - mx1) / pw) / _VAR1
    lth = jnp.log((my2 - my1) / ph) / _VAR1

    def sl1(d):
        a = jnp.abs(d)
        return jnp.where(a < 1.0, 0.5 * d * d, a - 0.5)

    l0 = sl1(locr_ref[0, 0] - ltx) + sl1(locr_ref[0, 1] - lty)
    l1 = sl1(locr_ref[0, 2] - ltw) + sl1(locr_ref[0, 3] - lth)
    loc_partial = jnp.sum((l0 + l1) * posf)

    x = conf_ref[0]
    sp = _softplus(x)
    pos_conf_partial = jnp.sum(posf * (sp - x))
    neg_clm = jnp.where(pos, 0.0, sp)

    k = jnp.minimum(jnp.sum(pos.astype(jnp.int32)) * _NEG_POS_RATE, _P)
    keys = neg_clm.view(jnp.int32)

    thr = jnp.int32(0)
    for bit in range(30, -1, -1):
        cand = thr | jnp.int32(1 << bit)
        cnt = jnp.sum((keys >= cand).astype(jnp.int32))
        thr = jnp.where(cnt >= k, cand, thr)
    above = keys > thr
    cnt_gt = jnp.sum(above.astype(jnp.int32))
    sum_gt = jnp.sum(jnp.where(above, neg_clm, 0.0))
    vk = thr.view(jnp.float32)
    neg_partial = jnp.where(
        k > 0, sum_gt + (k - cnt_gt).astype(jnp.float32) * vk, 0.0)

    lane = lax.broadcasted_iota(jnp.int32, (1, 128), 1)
    row = (jnp.where(lane == 0, loc_partial, 0.0)
           + jnp.where(lane == 1, pos_conf_partial, 0.0)
           + jnp.where(lane == 2, neg_partial, 0.0)
           + jnp.where(lane == 3, pos_cnt, 0.0))
    out_ref[0] = row


def kernel(loc_p, conf_p, priors, targets):
    B, P = conf_p.shape
    pcx, pcy = priors[:, 0], priors[:, 1]
    pww, phh = priors[:, 2], priors[:, 3]
    px1 = pcx - pww * 0.5
    py1 = pcy - phh * 0.5
    px2 = pcx + pww * 0.5
    py2 = pcy + phh * 0.5
    area = pww * phh
    pf = jnp.stack([pcx, pcy, pww, phh, px1, py1, px2, py2, area]
                   ).reshape(9, _R, _C)

    locr = loc_p.transpose(0, 2, 1).reshape(B, 4, _R, _C)
    confr = conf_p.reshape(B, _R, _C)

    out = pl.pallas_call(
        _loss_kernel,
        grid=(B,),
        in_specs=[
            pl.BlockSpec((1, _T, 5), lambda b: (b, 0, 0)),
            pl.BlockSpec((1, 4, _R, _C), lambda b: (b, 0, 0, 0)),
            pl.BlockSpec((1, _R, _C), lambda b: (b, 0, 0)),
            pl.BlockSpec((9, _R, _C), lambda b: (0, 0, 0)),
        ],
        out_specs=pl.BlockSpec((1, 1, 128), lambda b: (b, 0, 0)),
        out_shape=jax.ShapeDtypeStruct((B, 1, 128), jnp.float32),
        scratch_shapes=[
            pltpu.VMEM((_R, _C), jnp.float32),
            pltpu.VMEM((4, _R, _C), jnp.float32),
        ],
        compiler_params=pltpu.CompilerParams(
            dimension_semantics=("parallel",)),
    )(targets, locr, confr, pf)

    parts = jnp.sum(out[:, 0, 0:4], axis=0)
    pos_sum = jnp.maximum(parts[3], 1.0)
    loc_loss = parts[0] / pos_sum
    conf_loss = _CONF_GAIN * (parts[1] + parts[2]) / pos_sum
    return (loc_loss, conf_loss)


# speedup vs baseline: 115.6980x; 1.4350x over previous
import functools

import jax
import jax.numpy as jnp
from jax import lax
from jax.experimental import pallas as pl
from jax.experimental.pallas import tpu as pltpu

_MATCH_OVERLAP = 0.35
_NEG_POS_RATE = 3
_VAR0, _VAR1 = 0.1, 0.2
_CONF_GAIN = 1.0

_T = 32
_P = 32768
_R, _C = 8, 4096
_BIG_I32 = 2147483647
_BPP = 4


def _softplus(x):
    return jnp.maximum(x, 0.0) + jnp.log1p(jnp.exp(-jnp.abs(x)))


def _match_phase(t5, pf_ref, bto_ref, matched_ref):
    tx1 = t5[:, 0:1]
    ty1 = t5[:, 1:2]
    tx2 = t5[:, 2:3]
    ty2 = t5[:, 3:4]
    area_t = (tx2 - tx1) * (ty2 - ty1)
    truths4 = t5[:, 0:4]

    iota_t = lax.broadcasted_iota(jnp.int32, (_T, 1), 0)
    iota_c = lax.broadcasted_iota(jnp.int32, (1, _C), 1)

    run_max = jnp.full((_T, 1), -1.0, dtype=jnp.float32)
    run_idx = jnp.zeros((_T, 1), dtype=jnp.int32)

    for c in range(_R):
        px1 = pf_ref[4, c, :][None, :]
        py1 = pf_ref[5, c, :][None, :]
        px2 = pf_ref[6, c, :][None, :]
        py2 = pf_ref[7, c, :][None, :]
        area_p = pf_ref[8, c, :][None, :]

        iw = jnp.maximum(jnp.minimum(tx2, px2) - jnp.maximum(tx1, px1), 0.0)
        ih = jnp.maximum(jnp.minimum(ty2, py2) - jnp.maximum(ty1, py1), 0.0)
        inter = iw * ih
        iou = inter / (area_t + area_p - inter)

        bto_c = jnp.max(iou, axis=0, keepdims=True)
        bti_c = jnp.min(jnp.where(iou == bto_c, iota_t, _BIG_I32),
                        axis=0, keepdims=True)
        oh = (bti_c == iota_t).astype(jnp.float32)
        matched_c = lax.dot_general(truths4, oh, (((0,), (0,)), ((), ())),
                                    preferred_element_type=jnp.float32)
        bto_ref[c, :] = bto_c[0]
        matched_ref[:, c, :] = matched_c

        cmax = jnp.max(iou, axis=1, keepdims=True)
        carg = jnp.min(jnp.where(iou == cmax, iota_c, _BIG_I32),
                       axis=1, keepdims=True) + c * _C
        upd = cmax > run_max
        run_max = jnp.where(upd, cmax, run_max)
        run_idx = jnp.where(upd, carg, run_idx)

    return run_idx, truths4


def _override_step(t, run_idx, truths4, bto_ref, matched_ref, ri8, li8):
    p = run_idx[t, 0]
    r = p // _C
    l = p - r * _C
    lane_base = pl.multiple_of((l // 128) * 128, 128)
    l_off = l - lane_base
    m = (ri8 == r) & (li8 == l_off)
    tile = bto_ref[:, pl.ds(lane_base, 128)]
    bto_ref[:, pl.ds(lane_base, 128)] = jnp.where(m, 2.0, tile)
    mtile = matched_ref[:, :, pl.ds(lane_base, 128)]
    coords = truths4[t, :].reshape(4, 1, 1)
    matched_ref[:, :, pl.ds(lane_base, 128)] = jnp.where(
        m[None], coords, mtile)


def _phase_b(locr, conf, pf_ref, bto_ref, matched_ref):
    bto = bto_ref[:, :]
    pos = bto >= _MATCH_OVERLAP
    posf = pos.astype(jnp.float32)
    pos_cnt = jnp.sum(posf)

    cx = pf_ref[0]
    cy = pf_ref[1]
    pw = pf_ref[2]
    ph = pf_ref[3]
    mx1 = matched_ref[0]
    my1 = matched_ref[1]
    mx2 = matched_ref[2]
    my2 = matched_ref[3]

    ltx = ((mx1 + mx2) * 0.5 - cx) / (_VAR0 * pw)
    lty = ((my1 + my2) * 0.5 - cy) / (_VAR0 * ph)
    ltw = jnp.log((mx2 - mx1) / pw) / _VAR1
    lth = jnp.log((my2 - my1) / ph) / _VAR1

    def sl1(d):
        a = jnp.abs(d)
        return jnp.where(a < 1.0, 0.5 * d * d, a - 0.5)

    l0 = sl1(locr[0] - ltx) + sl1(locr[1] - lty)
    l1 = sl1(locr[2] - ltw) + sl1(locr[3] - lth)
    loc_partial = jnp.sum((l0 + l1) * posf)

    x = conf
    sp = _softplus(x)
    pos_conf_partial = jnp.sum(posf * (sp - x))
    neg_clm = jnp.where(pos, 0.0, sp)

    k = jnp.minimum(jnp.sum(pos.astype(jnp.int32)) * _NEG_POS_RATE, _P)
    return loc_partial, pos_conf_partial, neg_clm, pos_cnt, k


def _loss_kernel(targets_ref, locr_ref, conf_ref, pf_ref, out_ref,
                 *scratch):
    btos = scratch[0::2]
    matcheds = scratch[1::2]

    run_idx = [None] * _BPP
    truths4 = [None] * _BPP
    for b in range(_BPP):
        run_idx[b], truths4[b] = _match_phase(
            targets_ref[b], pf_ref, btos[b], matcheds[b])

    ri8 = lax.broadcasted_iota(jnp.int32, (8, 128), 0)
    li8 = lax.broadcasted_iota(jnp.int32, (8, 128), 1)
    for t in range(_T):
        for b in range(_BPP):
            _override_step(t, run_idx[b], truths4[b], btos[b], matcheds[b],
                           ri8, li8)

    loc_p = [None] * _BPP
    posc_p = [None] * _BPP
    neg_clm = [None] * _BPP
    pos_cnt = [None] * _BPP
    kk = [None] * _BPP
    for b in range(_BPP):
        loc_p[b], posc_p[b], neg_clm[b], pos_cnt[b], kk[b] = _phase_b(
            locr_ref[b], conf_ref[b], pf_ref, btos[b], matcheds[b])

    keys = [neg_clm[b].view(jnp.int32) for b in range(_BPP)]
    thr = [jnp.int32(0)] * _BPP
    for bit in range(30, -1, -1):
        for b in range(_BPP):
            cand = thr[b] | jnp.int32(1 << bit)
            cnt = jnp.sum((keys[b] >= cand).astype(jnp.int32))
            thr[b] = jnp.where(cnt >= kk[b], cand, thr[b])

    lane = lax.broadcasted_iota(jnp.int32, (1, 128), 1)
    for b in range(_BPP):
        above = keys[b] > thr[b]
        cnt_gt = jnp.sum(above.astype(jnp.int32))
        sum_gt = jnp.sum(jnp.where(above, neg_clm[b], 0.0))
        vk = thr[b].view(jnp.float32)
        neg_partial = jnp.where(
            kk[b] > 0,
            sum_gt + (kk[b] - cnt_gt).astype(jnp.float32) * vk, 0.0)
        row = (jnp.where(lane == 0, loc_p[b], 0.0)
               + jnp.where(lane == 1, posc_p[b], 0.0)
               + jnp.where(lane == 2, neg_partial, 0.0)
               + jnp.where(lane == 3, pos_cnt[b], 0.0))
        out_ref[0, b] = row[0]


def kernel(loc_p, conf_p, priors, targets):
    B, P = conf_p.shape
    pcx, pcy = priors[:, 0], priors[:, 1]
    pww, phh = priors[:, 2], priors[:, 3]
    px1 = pcx - pww * 0.5
    py1 = pcy - phh * 0.5
    px2 = pcx + pww * 0.5
    py2 = pcy + phh * 0.5
    area = pww * phh
    pf = jnp.stack([pcx, pcy, pww, phh, px1, py1, px2, py2, area]
                   ).reshape(9, _R, _C)

    locr = loc_p.transpose(0, 2, 1).reshape(B, 4, _R, _C)
    confr = conf_p.reshape(B, _R, _C)
    nprog = B // _BPP

    out = pl.pallas_call(
        _loss_kernel,
        grid=(nprog,),
        in_specs=[
            pl.BlockSpec((_BPP, _T, 5), lambda b: (b, 0, 0)),
            pl.BlockSpec((_BPP, 4, _R, _C), lambda b: (b, 0, 0, 0)),
            pl.BlockSpec((_BPP, _R, _C), lambda b: (b, 0, 0)),
            pl.BlockSpec((9, _R, _C), lambda b: (0, 0, 0)),
        ],
        out_specs=pl.BlockSpec((1, _BPP, 128), lambda b: (b, 0, 0)),
        out_shape=jax.ShapeDtypeStruct((nprog, _BPP, 128), jnp.float32),
        scratch_shapes=[
            shp
            for _ in range(_BPP)
            for shp in (pltpu.VMEM((_R, _C), jnp.float32),
                        pltpu.VMEM((4, _R, _C), jnp.float32))
        ],
        compiler_params=pltpu.CompilerParams(
            dimension_semantics=("parallel",)),
    )(targets, locr, confr, pf)

    parts = jnp.sum(out.reshape(B, 128)[:, 0:4], axis=0)
    pos_sum = jnp.maximum(parts[3], 1.0)
    loc_loss = parts[0] / pos_sum
    conf_loss = _CONF_GAIN * (parts[1] + parts[2]) / pos_sum
    return (loc_loss, conf_loss)
